# Initial kernel scaffold; baseline (speedup 1.0000x reference)
import functools

import numpy as np
import jax
import jax.numpy as jnp
from jax import lax
from jax.experimental import pallas as pl
from jax.experimental.pallas import tpu as pltpu
from jax.experimental.pallas import tpu_sc as plsc

_NUM_BUCKETS = 32
_MAX_DISTANCE = 128
_NUM_HEADS = 16
_Q_LEN = 2048
_K_LEN = 2048

_NDIAG = _Q_LEN + _K_LEN - 1
_DPAD = 4112
_NPHASE = 8
_NW = 32


def _bucket_per_diagonal() -> np.ndarray:
    d = np.arange(_DPAD, dtype=np.int64)
    rel = np.clip(d, 0, _NDIAG - 1) - (_Q_LEN - 1)
    n = -rel
    half = _NUM_BUCKETS // 2
    sign = (n < 0).astype(np.int32)
    n = np.abs(n)
    max_exact = half // 2
    nf = n.astype(np.float32)
    val_large = max_exact + (
        np.log(nf / max_exact + np.float32(1e-6))
        / np.log(np.float32(_MAX_DISTANCE / max_exact))
        * (half - max_exact)
    ).astype(np.int32)
    val_large = np.minimum(val_large, half - 1)
    bucket = np.where(n < max_exact, n.astype(np.int32), val_large)
    return (bucket + sign * half).astype(np.int32)


_BUCKET_STATIC = _bucket_per_diagonal()

_mesh = plsc.VectorSubcoreMesh(
    core_axis_name="c", subcore_axis_name="s", num_cores=2, num_subcores=16
)


@functools.partial(
    pl.kernel,
    out_type=jax.ShapeDtypeStruct((_NUM_HEADS, _Q_LEN, _K_LEN), jnp.float32),
    mesh=_mesh,
    scratch_types=[
        pltpu.VMEM((_NUM_HEADS, _NUM_BUCKETS), jnp.float32),
        pltpu.VMEM((_DPAD,), jnp.int32),
        pltpu.VMEM((_NPHASE * _DPAD,), jnp.float32),
    ],
)
def _bias_kernel(table_hbm, bucket_hbm, out_hbm, table_v, bucket_v, dp_v):
    wid = lax.axis_index("s") * 2 + lax.axis_index("c")
    h = wid // 2
    q0 = (wid % 2) * (_Q_LEN // 2)

    pltpu.sync_copy(table_hbm, table_v)
    pltpu.sync_copy(bucket_hbm, bucket_v)

    iota = lax.iota(jnp.int32, 16)
    hvec = jnp.full((16,), 0, dtype=jnp.int32) + h

    def build_phase(p, _):
        def body(j, _):
            idx = iota + (j * 16 + p)
            bk = plsc.load_gather(bucket_v, [idx])
            vals = plsc.load_gather(table_v, [hvec, bk])
            dp_v[pl.ds(p * _DPAD + j * 16, 16)] = vals
            return 0

        return lax.fori_loop(0, _DPAD // 16 - 1, body, 0)

    lax.fori_loop(0, _NPHASE, build_phase, 0)

    def row(i, _):
        q = q0 + i
        start = (_Q_LEN - 1) - q
        p = jnp.bitwise_and(start, 7)
        a = start - p
        pltpu.sync_copy(dp_v.at[pl.ds(p * _DPAD + a, _K_LEN)], out_hbm.at[h, q])
        return 0

    lax.fori_loop(0, _Q_LEN // 2, row, 0)


def kernel(bias_table, q_len, k_len):
    del q_len, k_len
    table_t = jnp.transpose(bias_table.astype(jnp.float32))
    bucket = jnp.asarray(_BUCKET_STATIC)
    out = _bias_kernel(table_t, bucket)
    return out[None]


# speedup vs baseline: 35.8250x; 35.8250x over previous
import functools

import numpy as np
import jax
import jax.numpy as jnp
from jax import lax
from jax.experimental import pallas as pl
from jax.experimental.pallas import tpu as pltpu
from jax.experimental.pallas import tpu_sc as plsc

_NUM_BUCKETS = 32
_MAX_DISTANCE = 128
_NUM_HEADS = 16
_Q_LEN = 2048
_K_LEN = 2048

_NDIAG = _Q_LEN + _K_LEN - 1
_DPAD = 4112
_NPHASE = 8


def _bucket_per_diagonal() -> np.ndarray:
    d = np.arange(_DPAD, dtype=np.int64)
    rel = np.clip(d, 0, _NDIAG - 1) - (_Q_LEN - 1)
    n = -rel
    half = _NUM_BUCKETS // 2
    sign = (n < 0).astype(np.int32)
    n = np.abs(n)
    max_exact = half // 2
    nf = n.astype(np.float32)
    val_large = max_exact + (
        np.log(nf / max_exact + np.float32(1e-6))
        / np.log(np.float32(_MAX_DISTANCE / max_exact))
        * (half - max_exact)
    ).astype(np.int32)
    val_large = np.minimum(val_large, half - 1)
    bucket = np.where(n < max_exact, n.astype(np.int32), val_large)
    return (bucket + sign * half).astype(np.int32)


_BUCKET_STATIC = _bucket_per_diagonal()


@functools.cache
def _build_bias_kernel():
    mesh = plsc.VectorSubcoreMesh(
        core_axis_name="c", subcore_axis_name="s", num_cores=2, num_subcores=16
    )
    return pl.kernel(
        _bias_kernel_body,
        out_type=jax.ShapeDtypeStruct((_NUM_HEADS, _Q_LEN, _K_LEN), jnp.float32),
        mesh=mesh,
        compiler_params=pltpu.CompilerParams(
            needs_layout_passes=False, use_tc_tiling_on_sc=False
        ),
        scratch_types=[
            pltpu.VMEM((_NUM_HEADS, _NUM_BUCKETS), jnp.float32),
            pltpu.VMEM((_DPAD,), jnp.int32),
            pltpu.VMEM((_NPHASE * _DPAD,), jnp.float32),
        ],
    )


def _bias_kernel_body(table_hbm, bucket_hbm, out_hbm, table_v, bucket_v, dp_v):
    wid = lax.axis_index("s") * 2 + lax.axis_index("c")
    h = wid // 2
    q0 = (wid % 2) * (_Q_LEN // 2)

    pltpu.sync_copy(table_hbm, table_v)
    pltpu.sync_copy(bucket_hbm, bucket_v)

    iota = lax.iota(jnp.int32, 16)
    hvec = jnp.zeros((16,), jnp.int32) + h

    def build_phase(p, _):
        def body(j, _):
            idx = iota + (j * 16 + p)
            bk = plsc.load_gather(bucket_v, [idx])
            vals = plsc.load_gather(table_v, [hvec, bk])
            dp_v[pl.ds(pl.multiple_of(p * _DPAD + j * 16, 8), 16)] = vals
            return 0

        return lax.fori_loop(0, _DPAD // 16 - 1, body, 0)

    lax.fori_loop(0, _NPHASE, build_phase, 0)

    def row(i, _):
        q = q0 + i
        start = (_Q_LEN - 1) - q
        p = jnp.bitwise_and(start, 7)
        a = start - p
        src = pl.ds(pl.multiple_of(p * _DPAD + a, 8), _K_LEN)
        pltpu.sync_copy(dp_v.at[src], out_hbm.at[h, q])
        return 0

    lax.fori_loop(0, _Q_LEN // 2, row, 0)


def kernel(bias_table, q_len, k_len):
    del q_len, k_len
    table_t = jnp.transpose(bias_table.astype(jnp.float32))
    bucket = jnp.asarray(_BUCKET_STATIC)
    out = _build_bias_kernel()(table_t, bucket)
    return out[None]


# speedup vs baseline: 36.0126x; 1.0052x over previous
import functools

import numpy as np
import jax
import jax.numpy as jnp
from jax import lax
from jax.experimental import pallas as pl
from jax.experimental.pallas import tpu as pltpu
from jax.experimental.pallas import tpu_sc as plsc

_NUM_BUCKETS = 32
_MAX_DISTANCE = 128
_NUM_HEADS = 16
_Q_LEN = 2048
_K_LEN = 2048

_NDIAG = _Q_LEN + _K_LEN - 1
_DPAD = 4112
_NPHASE = 8


def _bucket_per_diagonal() -> np.ndarray:
    d = np.arange(_DPAD, dtype=np.int64)
    rel = np.clip(d, 0, _NDIAG - 1) - (_Q_LEN - 1)
    n = -rel
    half = _NUM_BUCKETS // 2
    sign = (n < 0).astype(np.int32)
    n = np.abs(n)
    max_exact = half // 2
    nf = n.astype(np.float32)
    val_large = max_exact + (
        np.log(nf / max_exact + np.float32(1e-6))
        / np.log(np.float32(_MAX_DISTANCE / max_exact))
        * (half - max_exact)
    ).astype(np.int32)
    val_large = np.minimum(val_large, half - 1)
    bucket = np.where(n < max_exact, n.astype(np.int32), val_large)
    return (bucket + sign * half).astype(np.int32)


_BUCKET_STATIC = _bucket_per_diagonal()


@functools.cache
def _build_bias_kernel():
    mesh = plsc.VectorSubcoreMesh(
        core_axis_name="c", subcore_axis_name="s", num_cores=2, num_subcores=16
    )
    return pl.kernel(
        _bias_kernel_body,
        out_type=jax.ShapeDtypeStruct((1, _NUM_HEADS, _Q_LEN, _K_LEN), jnp.float32),
        mesh=mesh,
        compiler_params=pltpu.CompilerParams(
            needs_layout_passes=False, use_tc_tiling_on_sc=False
        ),
        scratch_types=[
            pltpu.VMEM((_NUM_HEADS, _NUM_BUCKETS), jnp.float32),
            pltpu.VMEM((_DPAD,), jnp.int32),
            pltpu.VMEM((_NPHASE * _DPAD,), jnp.float32),
        ],
    )


def _bias_kernel_body(table_hbm, bucket_hbm, out_hbm, table_v, bucket_v, dp_v):
    wid = lax.axis_index("s") * 2 + lax.axis_index("c")
    h = wid // 2
    q0 = (wid % 2) * (_Q_LEN // 2)

    pltpu.sync_copy(table_hbm, table_v)
    pltpu.sync_copy(bucket_hbm, bucket_v)

    iota = lax.iota(jnp.int32, 16)
    hvec = jnp.zeros((16,), jnp.int32) + h

    def build_phase(p, _):
        def body(j, _):
            idx = iota + (j * 16 + p)
            bk = plsc.load_gather(bucket_v, [idx])
            vals = plsc.load_gather(table_v, [hvec, bk])
            dp_v[pl.ds(pl.multiple_of(p * _DPAD + j * 16, 8), 16)] = vals
            return 0

        return lax.fori_loop(0, _DPAD // 16 - 1, body, 0)

    lax.fori_loop(0, _NPHASE, build_phase, 0)

    def row(i, _):
        q = q0 + i
        start = (_Q_LEN - 1) - q
        p = jnp.bitwise_and(start, 7)
        a = start - p
        src = pl.ds(pl.multiple_of(p * _DPAD + a, 8), _K_LEN)
        pltpu.sync_copy(dp_v.at[src], out_hbm.at[0, h, q])
        return 0

    lax.fori_loop(0, _Q_LEN // 2, row, 0)


def kernel(bias_table, q_len, k_len):
    del q_len, k_len
    table_t = jnp.transpose(bias_table.astype(jnp.float32))
    bucket = jnp.asarray(_BUCKET_STATIC)
    return _build_bias_kernel()(table_t, bucket)


# speedup vs baseline: 50.9239x; 1.4141x over previous
import functools

import numpy as np
import jax
import jax.numpy as jnp
from jax import lax
from jax.experimental import pallas as pl
from jax.experimental.pallas import tpu as pltpu
from jax.experimental.pallas import tpu_sc as plsc

_NUM_BUCKETS = 32
_MAX_DISTANCE = 128
_NUM_HEADS = 16
_Q_LEN = 2048
_K_LEN = 2048

_NDIAG = _Q_LEN + _K_LEN - 1
_D8W = 4224
_DPAD = 4240
_QBLK = 64


def _bucket_per_diagonal() -> np.ndarray:
    d = np.arange(_DPAD, dtype=np.int64)
    rel = np.clip(d, 0, _NDIAG - 1) - (_Q_LEN - 1)
    n = -rel
    half = _NUM_BUCKETS // 2
    sign = (n < 0).astype(np.int32)
    n = np.abs(n)
    max_exact = half // 2
    nf = n.astype(np.float32)
    val_large = max_exact + (
        np.log(nf / max_exact + np.float32(1e-6))
        / np.log(np.float32(_MAX_DISTANCE / max_exact))
        * (half - max_exact)
    ).astype(np.int32)
    val_large = np.minimum(val_large, half - 1)
    bucket = np.where(n < max_exact, n.astype(np.int32), val_large)
    return (bucket + sign * half).astype(np.int32)


_BUCKET_STATIC = _bucket_per_diagonal()


@functools.cache
def _build_d8_kernel():
    mesh = plsc.VectorSubcoreMesh(
        core_axis_name="c", subcore_axis_name="s", num_cores=2, num_subcores=16
    )
    return pl.kernel(
        _d8_kernel_body,
        out_type=jax.ShapeDtypeStruct((_NUM_HEADS, 8, _D8W), jnp.float32),
        mesh=mesh,
        compiler_params=pltpu.CompilerParams(
            needs_layout_passes=False, use_tc_tiling_on_sc=False
        ),
        scratch_types=[
            pltpu.VMEM((_NUM_HEADS * _NUM_BUCKETS,), jnp.float32),
            pltpu.VMEM((_DPAD,), jnp.int32),
            pltpu.VMEM((8 * _D8W,), jnp.float32),
        ],
    )


def _d8_kernel_body(table_hbm, bucket_hbm, d8_hbm, table_v, bucket_v, dp_v):
    wid = lax.axis_index("s") * 2 + lax.axis_index("c")
    h = wid // 2
    j0 = (wid % 2) * 4

    pltpu.sync_copy(table_hbm, table_v)
    pltpu.sync_copy(bucket_hbm, bucket_v)

    iota = lax.iota(jnp.int32, 16)
    hbase = jnp.zeros((16,), jnp.int32) + h * _NUM_BUCKETS

    def build_phase(p, _):
        def body(i, _):
            bk = plsc.load_gather(bucket_v, [iota + (16 * i + p)])
            vals = plsc.load_gather(table_v, [hbase + bk])
            dp_v[pl.ds(pl.multiple_of(p * _D8W + 16 * i, 8), 16)] = vals
            return 0

        return lax.fori_loop(0, _D8W // 16, body, 0)

    lax.fori_loop(0, 8, build_phase, 0)

    for t in range(4):
        j = j0 + t
        src = pl.ds(pl.multiple_of((7 - j) * _D8W, 8), _D8W)
        pltpu.sync_copy(dp_v.at[src], d8_hbm.at[h, j])


_LOADW = _K_LEN + 128


def _bias_tc_body(d8_ref, out_ref):
    qb = pl.program_id(1)
    for g in range(_QBLK // 8):
        u = (_Q_LEN // 8 - 1) - _QBLK // 8 * qb - g
        phi = jnp.bitwise_and(u, 15)
        a = u // 16
        val = d8_ref[0, :, pl.ds(pl.multiple_of(128 * a, 128), _LOADW)]
        shift = jnp.where(phi == 0, 0, _LOADW - 8 * phi)
        rolled = pltpu.roll(val, shift, 1)
        out_ref[0, 0, pl.ds(8 * g, 8), :] = rolled[:, :_K_LEN]


@functools.cache
def _build_tc_kernel():
    return pl.pallas_call(
        _bias_tc_body,
        grid=(_NUM_HEADS, _Q_LEN // _QBLK),
        in_specs=[
            pl.BlockSpec((1, 8, _D8W), lambda hh, qq: (hh, 0, 0)),
        ],
        out_specs=pl.BlockSpec((1, 1, _QBLK, _K_LEN), lambda hh, qq: (0, hh, qq, 0)),
        out_shape=jax.ShapeDtypeStruct((1, _NUM_HEADS, _Q_LEN, _K_LEN), jnp.float32),
    )


def kernel(bias_table, q_len, k_len):
    del q_len, k_len
    table_flat = jnp.transpose(bias_table.astype(jnp.float32)).reshape(-1)
    bucket = jnp.asarray(_BUCKET_STATIC)
    d8 = _build_d8_kernel()(table_flat, bucket)
    return _build_tc_kernel()(d8)


# speedup vs baseline: 72.7765x; 1.4291x over previous
import functools

import numpy as np
import jax
import jax.numpy as jnp
from jax import lax
from jax.experimental import pallas as pl
from jax.experimental.pallas import tpu as pltpu
from jax.experimental.pallas import tpu_sc as plsc

_NUM_BUCKETS = 32
_MAX_DISTANCE = 128
_NUM_HEADS = 16
_Q_LEN = 2048
_K_LEN = 2048

_NDIAG = _Q_LEN + _K_LEN - 1
_D8W = 4224
_DPAD = 4240
_QBLK = 128


def _bucket_per_diagonal() -> np.ndarray:
    d = np.arange(_DPAD, dtype=np.int64)
    rel = np.clip(d, 0, _NDIAG - 1) - (_Q_LEN - 1)
    n = -rel
    half = _NUM_BUCKETS // 2
    sign = (n < 0).astype(np.int32)
    n = np.abs(n)
    max_exact = half // 2
    nf = n.astype(np.float32)
    val_large = max_exact + (
        np.log(nf / max_exact + np.float32(1e-6))
        / np.log(np.float32(_MAX_DISTANCE / max_exact))
        * (half - max_exact)
    ).astype(np.int32)
    val_large = np.minimum(val_large, half - 1)
    bucket = np.where(n < max_exact, n.astype(np.int32), val_large)
    return (bucket + sign * half).astype(np.int32)


_BUCKET_STATIC = _bucket_per_diagonal()


@functools.cache
def _build_d8_kernel():
    mesh = plsc.VectorSubcoreMesh(
        core_axis_name="c", subcore_axis_name="s", num_cores=2, num_subcores=16
    )
    return pl.kernel(
        _d8_kernel_body,
        out_type=jax.ShapeDtypeStruct((_NUM_HEADS, 8, _D8W), jnp.float32),
        mesh=mesh,
        compiler_params=pltpu.CompilerParams(
            needs_layout_passes=False, use_tc_tiling_on_sc=False
        ),
        scratch_types=[
            pltpu.VMEM((_NUM_HEADS * _NUM_BUCKETS,), jnp.float32),
            pltpu.VMEM((_DPAD,), jnp.int32),
            pltpu.VMEM((8 * _D8W,), jnp.float32),
        ],
    )


def _d8_kernel_body(table_hbm, bucket_hbm, d8_hbm, table_v, bucket_v, dp_v):
    wid = lax.axis_index("s") * 2 + lax.axis_index("c")
    h = wid // 2
    j0 = (wid % 2) * 4

    pltpu.sync_copy(table_hbm, table_v)
    pltpu.sync_copy(bucket_hbm, bucket_v)

    iota = lax.iota(jnp.int32, 16)
    hbase = jnp.zeros((16,), jnp.int32) + h * _NUM_BUCKETS

    def build_phase(p, _):
        def body(i, _):
            bk = plsc.load_gather(bucket_v, [iota + (16 * i + p)])
            vals = plsc.load_gather(table_v, [hbase + bk])
            dp_v[pl.ds(pl.multiple_of(p * _D8W + 16 * i, 8), 16)] = vals
            return 0

        return lax.fori_loop(0, _D8W // 16, body, 0)

    lax.fori_loop(0, 8, build_phase, 0)

    for t in range(4):
        j = j0 + t
        src = pl.ds(pl.multiple_of((7 - j) * _D8W, 8), _D8W)
        pltpu.sync_copy(dp_v.at[src], d8_hbm.at[h, j])


_LOADW = _K_LEN + 128


def _bias_tc_body(d8_ref, out_ref):
    qb = pl.program_id(1)
    a128 = pl.multiple_of(128 * (_Q_LEN // 128 - 1 - qb), 128)
    val = d8_ref[0, :, pl.ds(a128, _LOADW)]
    for g in range(_QBLK // 8):
        off = 120 - 8 * g
        out_ref[0, 0, pl.ds(8 * g, 8), :] = val[:, off : off + _K_LEN]


@functools.cache
def _build_tc_kernel():
    return pl.pallas_call(
        _bias_tc_body,
        grid=(_NUM_HEADS, _Q_LEN // _QBLK),
        in_specs=[
            pl.BlockSpec((1, 8, _D8W), lambda hh, qq: (hh, 0, 0)),
        ],
        out_specs=pl.BlockSpec((1, 1, _QBLK, _K_LEN), lambda hh, qq: (0, hh, qq, 0)),
        out_shape=jax.ShapeDtypeStruct((1, _NUM_HEADS, _Q_LEN, _K_LEN), jnp.float32),
    )


def kernel(bias_table, q_len, k_len):
    del q_len, k_len
    table_flat = jnp.transpose(bias_table.astype(jnp.float32)).reshape(-1)
    bucket = jnp.asarray(_BUCKET_STATIC)
    d8 = _build_d8_kernel()(table_flat, bucket)
    return _build_tc_kernel()(d8)


# speedup vs baseline: 92.6205x; 1.2727x over previous
import functools

import numpy as np
import jax
import jax.numpy as jnp
from jax import lax
from jax.experimental import pallas as pl
from jax.experimental.pallas import tpu as pltpu
from jax.experimental.pallas import tpu_sc as plsc

_NUM_BUCKETS = 32
_MAX_DISTANCE = 128
_NUM_HEADS = 16
_Q_LEN = 2048
_K_LEN = 2048

_NDIAG = _Q_LEN + _K_LEN - 1
_D8W = 4224
_DPAD = 4240
_QBLK = 256


def _bucket_per_diagonal() -> np.ndarray:
    d = np.arange(_DPAD, dtype=np.int64)
    rel = np.clip(d, 0, _NDIAG - 1) - (_Q_LEN - 1)
    n = -rel
    half = _NUM_BUCKETS // 2
    sign = (n < 0).astype(np.int32)
    n = np.abs(n)
    max_exact = half // 2
    nf = n.astype(np.float32)
    val_large = max_exact + (
        np.log(nf / max_exact + np.float32(1e-6))
        / np.log(np.float32(_MAX_DISTANCE / max_exact))
        * (half - max_exact)
    ).astype(np.int32)
    val_large = np.minimum(val_large, half - 1)
    bucket = np.where(n < max_exact, n.astype(np.int32), val_large)
    return (bucket + sign * half).astype(np.int32)


_BUCKET_STATIC = _bucket_per_diagonal()


@functools.cache
def _build_d8_kernel():
    mesh = plsc.VectorSubcoreMesh(
        core_axis_name="c", subcore_axis_name="s", num_cores=2, num_subcores=16
    )
    return pl.kernel(
        _d8_kernel_body,
        out_type=jax.ShapeDtypeStruct((_NUM_HEADS, 8, _D8W), jnp.float32),
        mesh=mesh,
        compiler_params=pltpu.CompilerParams(
            needs_layout_passes=False, use_tc_tiling_on_sc=False
        ),
        scratch_types=[
            pltpu.VMEM((_NUM_HEADS * _NUM_BUCKETS,), jnp.float32),
            pltpu.VMEM((_DPAD,), jnp.int32),
            pltpu.VMEM((8 * _D8W,), jnp.float32),
        ],
    )


def _d8_kernel_body(table_hbm, bucket_hbm, d8_hbm, table_v, bucket_v, dp_v):
    wid = lax.axis_index("s") * 2 + lax.axis_index("c")
    h = wid // 2
    j0 = (wid % 2) * 4

    pltpu.sync_copy(table_hbm, table_v)
    pltpu.sync_copy(bucket_hbm, bucket_v)

    iota = lax.iota(jnp.int32, 16)
    hbase = jnp.zeros((16,), jnp.int32) + h * _NUM_BUCKETS

    def build_phase(p, _):
        def body(i, _):
            bk = plsc.load_gather(bucket_v, [iota + (16 * i + p)])
            vals = plsc.load_gather(table_v, [hbase + bk])
            dp_v[pl.ds(pl.multiple_of(p * _D8W + 16 * i, 8), 16)] = vals
            return 0

        return lax.fori_loop(0, _D8W // 16, body, 0)

    lax.fori_loop(0, 8, build_phase, 0)

    for t in range(4):
        j = j0 + t
        src = pl.ds(pl.multiple_of((7 - j) * _D8W, 8), _D8W)
        pltpu.sync_copy(dp_v.at[src], d8_hbm.at[h, j])


_LOADW = _K_LEN + 128


def _bias_tc_body(d8_ref, out_ref):
    qb = pl.program_id(1)
    for c in range(_QBLK // 128):
        a128 = pl.multiple_of(128 * (_Q_LEN // 128 - 1 - (_QBLK // 128) * qb - c), 128)
        val = d8_ref[0, :, pl.ds(a128, _LOADW)]
        for g in range(16):
            off = 120 - 8 * g
            out_ref[0, 0, pl.ds(128 * c + 8 * g, 8), :] = val[:, off : off + _K_LEN]


@functools.cache
def _build_tc_kernel():
    return pl.pallas_call(
        _bias_tc_body,
        grid=(_NUM_HEADS, _Q_LEN // _QBLK),
        in_specs=[
            pl.BlockSpec((1, 8, _D8W), lambda hh, qq: (hh, 0, 0)),
        ],
        out_specs=pl.BlockSpec((1, 1, _QBLK, _K_LEN), lambda hh, qq: (0, hh, qq, 0)),
        out_shape=jax.ShapeDtypeStruct((1, _NUM_HEADS, _Q_LEN, _K_LEN), jnp.float32),
    )


def kernel(bias_table, q_len, k_len):
    del q_len, k_len
    table_flat = jnp.transpose(bias_table.astype(jnp.float32)).reshape(-1)
    bucket = jnp.asarray(_BUCKET_STATIC)
    d8 = _build_d8_kernel()(table_flat, bucket)
    return _build_tc_kernel()(d8)


# speedup vs baseline: 107.7454x; 1.1633x over previous
import functools

import numpy as np
import jax
import jax.numpy as jnp
from jax import lax
from jax.experimental import pallas as pl
from jax.experimental.pallas import tpu as pltpu
from jax.experimental.pallas import tpu_sc as plsc

_NUM_BUCKETS = 32
_MAX_DISTANCE = 128
_NUM_HEADS = 16
_Q_LEN = 2048
_K_LEN = 2048

_NDIAG = _Q_LEN + _K_LEN - 1
_D8W = 4224
_DPAD = 4240
_QBLK = 512


def _bucket_per_diagonal() -> np.ndarray:
    d = np.arange(_DPAD, dtype=np.int64)
    rel = np.clip(d, 0, _NDIAG - 1) - (_Q_LEN - 1)
    n = -rel
    half = _NUM_BUCKETS // 2
    sign = (n < 0).astype(np.int32)
    n = np.abs(n)
    max_exact = half // 2
    nf = n.astype(np.float32)
    val_large = max_exact + (
        np.log(nf / max_exact + np.float32(1e-6))
        / np.log(np.float32(_MAX_DISTANCE / max_exact))
        * (half - max_exact)
    ).astype(np.int32)
    val_large = np.minimum(val_large, half - 1)
    bucket = np.where(n < max_exact, n.astype(np.int32), val_large)
    return (bucket + sign * half).astype(np.int32)


_BUCKET_STATIC = _bucket_per_diagonal()


@functools.cache
def _build_d8_kernel():
    mesh = plsc.VectorSubcoreMesh(
        core_axis_name="c", subcore_axis_name="s", num_cores=2, num_subcores=16
    )
    return pl.kernel(
        _d8_kernel_body,
        out_type=jax.ShapeDtypeStruct((_NUM_HEADS, 8, _D8W), jnp.float32),
        mesh=mesh,
        compiler_params=pltpu.CompilerParams(
            needs_layout_passes=False, use_tc_tiling_on_sc=False
        ),
        scratch_types=[
            pltpu.VMEM((_NUM_HEADS * _NUM_BUCKETS,), jnp.float32),
            pltpu.VMEM((_DPAD,), jnp.int32),
            pltpu.VMEM((8 * _D8W,), jnp.float32),
        ],
    )


def _d8_kernel_body(table_hbm, bucket_hbm, d8_hbm, table_v, bucket_v, dp_v):
    wid = lax.axis_index("s") * 2 + lax.axis_index("c")
    h = wid // 2
    j0 = (wid % 2) * 4

    pltpu.sync_copy(table_hbm, table_v)
    pltpu.sync_copy(bucket_hbm, bucket_v)

    iota = lax.iota(jnp.int32, 16)
    hbase = jnp.zeros((16,), jnp.int32) + h * _NUM_BUCKETS

    def build_phase(p, _):
        def body(i, _):
            bk = plsc.load_gather(bucket_v, [iota + (16 * i + p)])
            vals = plsc.load_gather(table_v, [hbase + bk])
            dp_v[pl.ds(pl.multiple_of(p * _D8W + 16 * i, 8), 16)] = vals
            return 0

        return lax.fori_loop(0, _D8W // 16, body, 0)

    lax.fori_loop(0, 8, build_phase, 0)

    for t in range(4):
        j = j0 + t
        src = pl.ds(pl.multiple_of((7 - j) * _D8W, 8), _D8W)
        pltpu.sync_copy(dp_v.at[src], d8_hbm.at[h, j])


_LOADW = _K_LEN + 128


def _bias_tc_body(d8_ref, out_ref):
    qb = pl.program_id(1)
    for c in range(_QBLK // 128):
        a128 = pl.multiple_of(128 * (_Q_LEN // 128 - 1 - (_QBLK // 128) * qb - c), 128)
        val = d8_ref[0, :, pl.ds(a128, _LOADW)]
        for g in range(16):
            off = 120 - 8 * g
            out_ref[0, 0, pl.ds(128 * c + 8 * g, 8), :] = val[:, off : off + _K_LEN]


@functools.cache
def _build_tc_kernel():
    return pl.pallas_call(
        _bias_tc_body,
        grid=(_NUM_HEADS, _Q_LEN // _QBLK),
        in_specs=[
            pl.BlockSpec((1, 8, _D8W), lambda hh, qq: (hh, 0, 0)),
        ],
        out_specs=pl.BlockSpec((1, 1, _QBLK, _K_LEN), lambda hh, qq: (0, hh, qq, 0)),
        out_shape=jax.ShapeDtypeStruct((1, _NUM_HEADS, _Q_LEN, _K_LEN), jnp.float32),
    )


def kernel(bias_table, q_len, k_len):
    del q_len, k_len
    table_flat = jnp.transpose(bias_table.astype(jnp.float32)).reshape(-1)
    bucket = jnp.asarray(_BUCKET_STATIC)
    d8 = _build_d8_kernel()(table_flat, bucket)
    return _build_tc_kernel()(d8)


# speedup vs baseline: 116.6386x; 1.0825x over previous
import functools

import numpy as np
import jax
import jax.numpy as jnp
from jax import lax
from jax.experimental import pallas as pl
from jax.experimental.pallas import tpu as pltpu
from jax.experimental.pallas import tpu_sc as plsc

_NUM_BUCKETS = 32
_MAX_DISTANCE = 128
_NUM_HEADS = 16
_Q_LEN = 2048
_K_LEN = 2048

_NDIAG = _Q_LEN + _K_LEN - 1
_D8W = 4224
_DPAD = 4240
_QBLK = 1024


def _bucket_per_diagonal() -> np.ndarray:
    d = np.arange(_DPAD, dtype=np.int64)
    rel = np.clip(d, 0, _NDIAG - 1) - (_Q_LEN - 1)
    n = -rel
    half = _NUM_BUCKETS // 2
    sign = (n < 0).astype(np.int32)
    n = np.abs(n)
    max_exact = half // 2
    nf = n.astype(np.float32)
    val_large = max_exact + (
        np.log(nf / max_exact + np.float32(1e-6))
        / np.log(np.float32(_MAX_DISTANCE / max_exact))
        * (half - max_exact)
    ).astype(np.int32)
    val_large = np.minimum(val_large, half - 1)
    bucket = np.where(n < max_exact, n.astype(np.int32), val_large)
    return (bucket + sign * half).astype(np.int32)


_BUCKET_STATIC = _bucket_per_diagonal()


@functools.cache
def _build_d8_kernel():
    mesh = plsc.VectorSubcoreMesh(
        core_axis_name="c", subcore_axis_name="s", num_cores=2, num_subcores=16
    )
    return pl.kernel(
        _d8_kernel_body,
        out_type=jax.ShapeDtypeStruct((_NUM_HEADS, 8, _D8W), jnp.float32),
        mesh=mesh,
        compiler_params=pltpu.CompilerParams(
            needs_layout_passes=False, use_tc_tiling_on_sc=False
        ),
        scratch_types=[
            pltpu.VMEM((_NUM_HEADS * _NUM_BUCKETS,), jnp.float32),
            pltpu.VMEM((_DPAD,), jnp.int32),
            pltpu.VMEM((8 * _D8W,), jnp.float32),
        ],
    )


def _d8_kernel_body(table_hbm, bucket_hbm, d8_hbm, table_v, bucket_v, dp_v):
    wid = lax.axis_index("s") * 2 + lax.axis_index("c")
    h = wid // 2
    j0 = (wid % 2) * 4

    pltpu.sync_copy(table_hbm, table_v)
    pltpu.sync_copy(bucket_hbm, bucket_v)

    iota = lax.iota(jnp.int32, 16)
    hbase = jnp.zeros((16,), jnp.int32) + h * _NUM_BUCKETS

    def build_phase(p, _):
        def body(i, _):
            bk = plsc.load_gather(bucket_v, [iota + (16 * i + p)])
            vals = plsc.load_gather(table_v, [hbase + bk])
            dp_v[pl.ds(pl.multiple_of(p * _D8W + 16 * i, 8), 16)] = vals
            return 0

        return lax.fori_loop(0, _D8W // 16, body, 0)

    lax.fori_loop(0, 8, build_phase, 0)

    for t in range(4):
        j = j0 + t
        src = pl.ds(pl.multiple_of((7 - j) * _D8W, 8), _D8W)
        pltpu.sync_copy(dp_v.at[src], d8_hbm.at[h, j])


_LOADW = _K_LEN + 128


def _bias_tc_body(d8_ref, out_ref):
    qb = pl.program_id(1)
    for c in range(_QBLK // 128):
        a128 = pl.multiple_of(128 * (_Q_LEN // 128 - 1 - (_QBLK // 128) * qb - c), 128)
        val = d8_ref[0, :, pl.ds(a128, _LOADW)]
        for g in range(16):
            off = 120 - 8 * g
            out_ref[0, 0, pl.ds(128 * c + 8 * g, 8), :] = val[:, off : off + _K_LEN]


@functools.cache
def _build_tc_kernel():
    return pl.pallas_call(
        _bias_tc_body,
        grid=(_NUM_HEADS, _Q_LEN // _QBLK),
        in_specs=[
            pl.BlockSpec((1, 8, _D8W), lambda hh, qq: (hh, 0, 0)),
        ],
        out_specs=pl.BlockSpec((1, 1, _QBLK, _K_LEN), lambda hh, qq: (0, hh, qq, 0)),
        out_shape=jax.ShapeDtypeStruct((1, _NUM_HEADS, _Q_LEN, _K_LEN), jnp.float32),
    )


def kernel(bias_table, q_len, k_len):
    del q_len, k_len
    table_flat = jnp.transpose(bias_table.astype(jnp.float32)).reshape(-1)
    bucket = jnp.asarray(_BUCKET_STATIC)
    d8 = _build_d8_kernel()(table_flat, bucket)
    return _build_tc_kernel()(d8)


# speedup vs baseline: 118.3395x; 1.0146x over previous
import functools

import numpy as np
import jax
import jax.numpy as jnp
from jax import lax
from jax.experimental import pallas as pl
from jax.experimental.pallas import tpu as pltpu
from jax.experimental.pallas import tpu_sc as plsc

_NUM_BUCKETS = 32
_MAX_DISTANCE = 128
_NUM_HEADS = 16
_Q_LEN = 2048
_K_LEN = 2048

_NDIAG = _Q_LEN + _K_LEN - 1
_D8W = 4224
_DPAD = 4240
_QBLK = 2048


def _bucket_per_diagonal() -> np.ndarray:
    d = np.arange(_DPAD, dtype=np.int64)
    rel = np.clip(d, 0, _NDIAG - 1) - (_Q_LEN - 1)
    n = -rel
    half = _NUM_BUCKETS // 2
    sign = (n < 0).astype(np.int32)
    n = np.abs(n)
    max_exact = half // 2
    nf = n.astype(np.float32)
    val_large = max_exact + (
        np.log(nf / max_exact + np.float32(1e-6))
        / np.log(np.float32(_MAX_DISTANCE / max_exact))
        * (half - max_exact)
    ).astype(np.int32)
    val_large = np.minimum(val_large, half - 1)
    bucket = np.where(n < max_exact, n.astype(np.int32), val_large)
    return (bucket + sign * half).astype(np.int32)


_BUCKET_STATIC = _bucket_per_diagonal()


@functools.cache
def _build_d8_kernel():
    mesh = plsc.VectorSubcoreMesh(
        core_axis_name="c", subcore_axis_name="s", num_cores=2, num_subcores=16
    )
    return pl.kernel(
        _d8_kernel_body,
        out_type=jax.ShapeDtypeStruct((_NUM_HEADS, 8, _D8W), jnp.float32),
        mesh=mesh,
        compiler_params=pltpu.CompilerParams(
            needs_layout_passes=False, use_tc_tiling_on_sc=False
        ),
        scratch_types=[
            pltpu.VMEM((_NUM_HEADS * _NUM_BUCKETS,), jnp.float32),
            pltpu.VMEM((_DPAD,), jnp.int32),
            pltpu.VMEM((8 * _D8W,), jnp.float32),
        ],
    )


def _d8_kernel_body(table_hbm, bucket_hbm, d8_hbm, table_v, bucket_v, dp_v):
    wid = lax.axis_index("s") * 2 + lax.axis_index("c")
    h = wid // 2
    j0 = (wid % 2) * 4

    pltpu.sync_copy(table_hbm, table_v)
    pltpu.sync_copy(bucket_hbm, bucket_v)

    iota = lax.iota(jnp.int32, 16)
    hbase = jnp.zeros((16,), jnp.int32) + h * _NUM_BUCKETS

    def build_phase(p, _):
        def body(i, _):
            bk = plsc.load_gather(bucket_v, [iota + (16 * i + p)])
            vals = plsc.load_gather(table_v, [hbase + bk])
            dp_v[pl.ds(pl.multiple_of(p * _D8W + 16 * i, 8), 16)] = vals
            return 0

        return lax.fori_loop(0, _D8W // 16, body, 0)

    lax.fori_loop(0, 8, build_phase, 0)

    for t in range(4):
        j = j0 + t
        src = pl.ds(pl.multiple_of((7 - j) * _D8W, 8), _D8W)
        pltpu.sync_copy(dp_v.at[src], d8_hbm.at[h, j])


_LOADW = _K_LEN + 128


def _bias_tc_body(d8_ref, out_ref):
    qb = pl.program_id(1)
    for c in range(_QBLK // 128):
        a128 = pl.multiple_of(128 * (_Q_LEN // 128 - 1 - (_QBLK // 128) * qb - c), 128)
        val = d8_ref[0, :, pl.ds(a128, _LOADW)]
        for g in range(16):
            off = 120 - 8 * g
            out_ref[0, 0, pl.ds(128 * c + 8 * g, 8), :] = val[:, off : off + _K_LEN]


@functools.cache
def _build_tc_kernel():
    return pl.pallas_call(
        _bias_tc_body,
        grid=(_NUM_HEADS, _Q_LEN // _QBLK),
        in_specs=[
            pl.BlockSpec((1, 8, _D8W), lambda hh, qq: (hh, 0, 0)),
        ],
        out_specs=pl.BlockSpec((1, 1, _QBLK, _K_LEN), lambda hh, qq: (0, hh, qq, 0)),
        out_shape=jax.ShapeDtypeStruct((1, _NUM_HEADS, _Q_LEN, _K_LEN), jnp.float32),
    )


def kernel(bias_table, q_len, k_len):
    del q_len, k_len
    table_flat = jnp.transpose(bias_table.astype(jnp.float32)).reshape(-1)
    bucket = jnp.asarray(_BUCKET_STATIC)
    d8 = _build_d8_kernel()(table_flat, bucket)
    return _build_tc_kernel()(d8)


# speedup vs baseline: 137.2711x; 1.1600x over previous
import functools

import numpy as np
import jax
import jax.numpy as jnp
from jax import lax
from jax.experimental import pallas as pl
from jax.experimental.pallas import tpu as pltpu
from jax.experimental.pallas import tpu_sc as plsc

_NUM_BUCKETS = 32
_MAX_DISTANCE = 128
_NUM_HEADS = 16
_Q_LEN = 2048
_K_LEN = 2048

_NDIAG = _Q_LEN + _K_LEN - 1
_D8W = 4224
_DPAD = 4240
_QBLK = 2048


def _bucket_per_diagonal() -> np.ndarray:
    d = np.arange(_DPAD, dtype=np.int64)
    rel = np.clip(d, 0, _NDIAG - 1) - (_Q_LEN - 1)
    n = -rel
    half = _NUM_BUCKETS // 2
    sign = (n < 0).astype(np.int32)
    n = np.abs(n)
    max_exact = half // 2
    nf = n.astype(np.float32)
    val_large = max_exact + (
        np.log(nf / max_exact + np.float32(1e-6))
        / np.log(np.float32(_MAX_DISTANCE / max_exact))
        * (half - max_exact)
    ).astype(np.int32)
    val_large = np.minimum(val_large, half - 1)
    bucket = np.where(n < max_exact, n.astype(np.int32), val_large)
    return (bucket + sign * half).astype(np.int32)


_BUCKET_STATIC = _bucket_per_diagonal()


@functools.cache
def _build_d8_kernel():
    mesh = plsc.VectorSubcoreMesh(
        core_axis_name="c", subcore_axis_name="s", num_cores=2, num_subcores=16
    )
    return pl.kernel(
        _d8_kernel_body,
        out_type=jax.ShapeDtypeStruct((_NUM_HEADS, 8, _D8W), jnp.float32),
        mesh=mesh,
        compiler_params=pltpu.CompilerParams(
            needs_layout_passes=False, use_tc_tiling_on_sc=False
        ),
        scratch_types=[
            pltpu.VMEM((_NUM_HEADS * _NUM_BUCKETS,), jnp.float32),
            pltpu.VMEM((_DPAD,), jnp.int32),
            pltpu.VMEM((8 * _D8W,), jnp.float32),
        ],
    )


def _d8_kernel_body(table_hbm, bucket_hbm, d8_hbm, table_v, bucket_v, dp_v):
    wid = lax.axis_index("s") * 2 + lax.axis_index("c")
    h = wid // 2
    j0 = (wid % 2) * 4

    pltpu.sync_copy(table_hbm, table_v)
    pltpu.sync_copy(bucket_hbm, bucket_v)

    iota = lax.iota(jnp.int32, 16)
    hbase = jnp.zeros((16,), jnp.int32) + h * _NUM_BUCKETS
    p0 = 4 - j0

    def body(i, _):
        base = 16 * i
        bk = bucket_v[pl.ds(pl.multiple_of(base, 8), 16)]
        vals = plsc.load_gather(table_v, [hbase + bk])
        for t in range(4):
            p = p0 + t
            idx = iota + (p * _D8W + base - p)
            plsc.store_scatter(dp_v, [idx], vals, mask=iota + (base - p) >= 0)
        return 0

    lax.fori_loop(0, _D8W // 16, body, 0)

    for t in range(4):
        j = j0 + t
        src = pl.ds(pl.multiple_of((7 - j) * _D8W, 8), _D8W)
        pltpu.sync_copy(dp_v.at[src], d8_hbm.at[h, j])


_LOADW = _K_LEN + 128


def _bias_tc_body(d8_ref, out_ref):
    qb = pl.program_id(1)
    for c in range(_QBLK // 128):
        a128 = pl.multiple_of(128 * (_Q_LEN // 128 - 1 - (_QBLK // 128) * qb - c), 128)
        val = d8_ref[0, :, pl.ds(a128, _LOADW)]
        for g in range(16):
            off = 120 - 8 * g
            out_ref[0, 0, pl.ds(128 * c + 8 * g, 8), :] = val[:, off : off + _K_LEN]


@functools.cache
def _build_tc_kernel():
    return pl.pallas_call(
        _bias_tc_body,
        grid=(_NUM_HEADS, _Q_LEN // _QBLK),
        in_specs=[
            pl.BlockSpec((1, 8, _D8W), lambda hh, qq: (hh, 0, 0)),
        ],
        out_specs=pl.BlockSpec((1, 1, _QBLK, _K_LEN), lambda hh, qq: (0, hh, qq, 0)),
        out_shape=jax.ShapeDtypeStruct((1, _NUM_HEADS, _Q_LEN, _K_LEN), jnp.float32),
    )


def kernel(bias_table, q_len, k_len):
    del q_len, k_len
    table_flat = jnp.transpose(bias_table.astype(jnp.float32)).reshape(-1)
    bucket = jnp.asarray(_BUCKET_STATIC)
    d8 = _build_d8_kernel()(table_flat, bucket)
    return _build_tc_kernel()(d8)
